# baseline (speedup 1.0000x reference)
import jax
import jax.numpy as jnp
from jax import lax
from jax.experimental import pallas as pl
from jax.experimental.pallas import tpu as pltpu
from jax.experimental.pallas import tpu_sc as plsc

N_NODES = 10000
N_EDGES = 320000
NUM_SUBCORES = 16
N_PAD = 10240
BINS_PER_TILE = N_PAD // NUM_SUBCORES
LANES = 16
BLOCK = 8 * LANES
NUM_BLOCKS = N_EDGES // BLOCK
BLOCKS_PER_TILE = NUM_BLOCKS // NUM_SUBCORES
EXTRA_BLOCKS = NUM_BLOCKS - BLOCKS_PER_TILE * NUM_SUBCORES
E_MAIN = BLOCKS_PER_TILE * BLOCK
NCHUNK = 4
BLOCKS_PER_CHUNK = BLOCKS_PER_TILE // NCHUNK
E_CHUNK = E_MAIN // NCHUNK


def _sc_body(edge_hbm, out_hbm, idx_v, extra_v, hist_v, red_v, col_v, sum_v,
             sems, sem2):
    s = lax.axis_index("s")

    with jax.named_scope("stage"):
        copies = [
            pltpu.async_copy(
                edge_hbm.at[:, pl.ds(s * E_MAIN + k * E_CHUNK, E_CHUNK)],
                idx_v.at[:, pl.ds(k * E_CHUNK, E_CHUNK)], sems[k])
            for k in range(NCHUNK)
        ]
        tail = pltpu.async_copy(
            edge_hbm.at[:, pl.ds((BLOCKS_PER_TILE * NUM_SUBCORES
                                  + jnp.minimum(s, EXTRA_BLOCKS - 1)) * BLOCK,
                                 BLOCK)],
            extra_v, sem2)

        zero = jnp.zeros((LANES,), dtype=jnp.float32)

        def fill_zero(i, _):
            for u in range(8):
                hist_v[pl.ds((i * 8 + u) * LANES, LANES)] = zero
            return 0

        lax.fori_loop(0, N_PAD // (8 * LANES), fill_zero, 0)

    one = jnp.full((LANES,), 1.0, dtype=jnp.float32)

    with jax.named_scope("accum"):
        def accum(i, _):
            idxs = [idx_v[0, pl.ds(i * BLOCK + u * LANES, LANES)]
                    for u in range(8)]
            for idx in idxs:
                plsc.addupdate_scatter(hist_v, [idx], one)
            return 0

        for k in range(NCHUNK):
            copies[k].wait()
            lax.fori_loop(k * BLOCKS_PER_CHUNK, (k + 1) * BLOCKS_PER_CHUNK,
                          accum, 0)

        tail.wait()

        @pl.when(s < EXTRA_BLOCKS)
        def _():
            idxs = [extra_v[0, pl.ds(u * LANES, LANES)] for u in range(8)]
            for idx in idxs:
                plsc.addupdate_scatter(hist_v, [idx], one)

    with jax.named_scope("publish"):
        pltpu.sync_copy(hist_v, red_v.at[s])
        plsc.subcore_barrier()

    with jax.named_scope("gather"):
        pltpu.sync_copy(red_v.at[:, pl.ds(s * BINS_PER_TILE, BINS_PER_TILE)],
                        col_v)

    with jax.named_scope("reduce"):
        def reduce_vec(j, _):
            acc = one
            for r in range(NUM_SUBCORES):
                acc = acc + col_v[r, pl.ds(j * LANES, LANES)]
            sum_v[pl.ds(j * LANES, LANES)] = acc
            return 0

        lax.fori_loop(0, BINS_PER_TILE // LANES, reduce_vec, 0)

    with jax.named_scope("writeback"):
        last = N_NODES - (NUM_SUBCORES - 1) * BINS_PER_TILE

        @pl.when(s < NUM_SUBCORES - 1)
        def _():
            pltpu.sync_copy(
                sum_v.at[pl.ds(0, BINS_PER_TILE)],
                out_hbm.at[pl.ds(s * BINS_PER_TILE, BINS_PER_TILE)])

        @pl.when(s == NUM_SUBCORES - 1)
        def _():
            pltpu.sync_copy(
                sum_v.at[pl.ds(0, last)],
                out_hbm.at[pl.ds((NUM_SUBCORES - 1) * BINS_PER_TILE, last)])


@jax.jit
def _degree(edge_index):
    mesh = plsc.VectorSubcoreMesh(
        core_axis_name="c", subcore_axis_name="s",
        num_cores=1, num_subcores=NUM_SUBCORES)
    return pl.kernel(
        _sc_body,
        out_type=jax.ShapeDtypeStruct((N_NODES,), jnp.float32),
        mesh=mesh,
        scratch_types=[
            pltpu.VMEM((2, E_MAIN), jnp.int32),
            pltpu.VMEM((2, BLOCK), jnp.int32),
            pltpu.VMEM((N_PAD,), jnp.float32),
            pltpu.VMEM_SHARED((NUM_SUBCORES, N_PAD), jnp.float32),
            pltpu.VMEM((NUM_SUBCORES, BINS_PER_TILE), jnp.float32),
            pltpu.VMEM((BINS_PER_TILE,), jnp.float32),
            [pltpu.SemaphoreType.DMA] * NCHUNK,
            pltpu.SemaphoreType.DMA,
        ],
        compiler_params=pltpu.CompilerParams(needs_layout_passes=False),
    )(edge_index)


def kernel(edge_index, x):
    return _degree(edge_index)


# speedup vs baseline: 1.0133x; 1.0133x over previous
import jax
import jax.numpy as jnp
from jax import lax
from jax.experimental import pallas as pl
from jax.experimental.pallas import tpu as pltpu
from jax.experimental.pallas import tpu_sc as plsc

N_NODES = 10000
N_EDGES = 320000
NUM_SUBCORES = 16
N_PAD = 10240
BINS_PER_TILE = N_PAD // NUM_SUBCORES
LANES = 16
BLOCK = 8 * LANES
NUM_BLOCKS = N_EDGES // BLOCK
BLOCKS_PER_TILE = NUM_BLOCKS // NUM_SUBCORES
EXTRA_BLOCKS = NUM_BLOCKS - BLOCKS_PER_TILE * NUM_SUBCORES
E_MAIN = BLOCKS_PER_TILE * BLOCK
NCHUNK = 2
BLOCKS_PER_CHUNK = BLOCKS_PER_TILE // NCHUNK
E_CHUNK = E_MAIN // NCHUNK


def _sc_body(edge_hbm, out_hbm, idx_v, extra_v, hist_v, red_v, col_v, sum_v,
             sems, sem2):
    s = lax.axis_index("s")

    with jax.named_scope("stage"):
        copies = [
            pltpu.async_copy(
                edge_hbm.at[:, pl.ds(s * E_MAIN + k * E_CHUNK, E_CHUNK)],
                idx_v.at[:, pl.ds(k * E_CHUNK, E_CHUNK)], sems[k])
            for k in range(NCHUNK)
        ]
        tail = pltpu.async_copy(
            edge_hbm.at[:, pl.ds((BLOCKS_PER_TILE * NUM_SUBCORES
                                  + jnp.minimum(s, EXTRA_BLOCKS - 1)) * BLOCK,
                                 BLOCK)],
            extra_v, sem2)

        zero = jnp.zeros((LANES,), dtype=jnp.float32)

        def fill_zero(i, _):
            for u in range(8):
                hist_v[pl.ds((i * 8 + u) * LANES, LANES)] = zero
            return 0

        lax.fori_loop(0, N_PAD // (8 * LANES), fill_zero, 0)

    one = jnp.full((LANES,), 1.0, dtype=jnp.float32)

    with jax.named_scope("accum"):
        def accum(i, _):
            idxs = [idx_v[0, pl.ds(i * BLOCK + u * LANES, LANES)]
                    for u in range(8)]
            for idx in idxs:
                plsc.addupdate_scatter(hist_v, [idx], one)
            return 0

        for k in range(NCHUNK):
            copies[k].wait()
            lax.fori_loop(k * BLOCKS_PER_CHUNK, (k + 1) * BLOCKS_PER_CHUNK,
                          accum, 0)

        tail.wait()

        @pl.when(s < EXTRA_BLOCKS)
        def _():
            idxs = [extra_v[0, pl.ds(u * LANES, LANES)] for u in range(8)]
            for idx in idxs:
                plsc.addupdate_scatter(hist_v, [idx], one)

    with jax.named_scope("publish"):
        pltpu.sync_copy(hist_v, red_v.at[s])
        plsc.subcore_barrier()

    with jax.named_scope("gather"):
        pltpu.sync_copy(red_v.at[:, pl.ds(s * BINS_PER_TILE, BINS_PER_TILE)],
                        col_v)

    with jax.named_scope("reduce"):
        def reduce_vec(j, _):
            acc = one
            for r in range(NUM_SUBCORES):
                acc = acc + col_v[r, pl.ds(j * LANES, LANES)]
            sum_v[pl.ds(j * LANES, LANES)] = acc
            return 0

        lax.fori_loop(0, BINS_PER_TILE // LANES, reduce_vec, 0)

    with jax.named_scope("writeback"):
        last = N_NODES - (NUM_SUBCORES - 1) * BINS_PER_TILE

        @pl.when(s < NUM_SUBCORES - 1)
        def _():
            pltpu.sync_copy(
                sum_v.at[pl.ds(0, BINS_PER_TILE)],
                out_hbm.at[pl.ds(s * BINS_PER_TILE, BINS_PER_TILE)])

        @pl.when(s == NUM_SUBCORES - 1)
        def _():
            pltpu.sync_copy(
                sum_v.at[pl.ds(0, last)],
                out_hbm.at[pl.ds((NUM_SUBCORES - 1) * BINS_PER_TILE, last)])


@jax.jit
def _degree(edge_index):
    mesh = plsc.VectorSubcoreMesh(
        core_axis_name="c", subcore_axis_name="s",
        num_cores=1, num_subcores=NUM_SUBCORES)
    return pl.kernel(
        _sc_body,
        out_type=jax.ShapeDtypeStruct((N_NODES,), jnp.float32),
        mesh=mesh,
        scratch_types=[
            pltpu.VMEM((2, E_MAIN), jnp.int32),
            pltpu.VMEM((2, BLOCK), jnp.int32),
            pltpu.VMEM((N_PAD,), jnp.float32),
            pltpu.VMEM_SHARED((NUM_SUBCORES, N_PAD), jnp.float32),
            pltpu.VMEM((NUM_SUBCORES, BINS_PER_TILE), jnp.float32),
            pltpu.VMEM((BINS_PER_TILE,), jnp.float32),
            [pltpu.SemaphoreType.DMA] * NCHUNK,
            pltpu.SemaphoreType.DMA,
        ],
        compiler_params=pltpu.CompilerParams(needs_layout_passes=False),
    )(edge_index)


def kernel(edge_index, x):
    return _degree(edge_index)
